# Initial kernel scaffold; baseline (speedup 1.0000x reference)
import jax
import jax.numpy as jnp
from jax.experimental import pallas as pl


def kernel(tokens, table):
    raise NotImplementedError("write your pallas kernel here")


# speedup vs baseline: 7.1207x; 7.1207x over previous
import functools

import jax
import jax.numpy as jnp
from jax import lax
from jax.experimental import pallas as pl
from jax.experimental.pallas import tpu as pltpu
from jax.experimental.pallas import tpu_sc as plsc

VOCAB = 50257
D = 128
NC = 2
NS = 16
NW = NC * NS

B = 16384 * 200
CH = 128
NCHUNK = B // CH
CPW = NCHUNK // NW
IK = 16
NGROUP = CPW // IK


def _body(tok_hbm, tab_hbm, out_hbm, idx_v, rows_v, gsem):
    wid = lax.axis_index("s") * NC + lax.axis_index("c")
    chunk0 = wid * CPW

    def group(g, carry):
        gchunk = chunk0 + g * IK
        pltpu.sync_copy(tok_hbm.at[pl.ds(gchunk, IK)], idx_v)
        for j in range(IK):
            pltpu.async_copy(tab_hbm.at[idx_v.at[j]], rows_v, gsem).wait()
            pltpu.sync_copy(rows_v, out_hbm.at[pl.ds((gchunk + j) * CH, CH)])
        return carry

    lax.fori_loop(0, NGROUP, group, 0)


@jax.jit
def _embed(tokens2d, table):
    kern = pl.kernel(
        _body,
        out_type=jax.ShapeDtypeStruct((B, D), jnp.float32),
        mesh=plsc.VectorSubcoreMesh(
            core_axis_name="c", subcore_axis_name="s",
            num_cores=NC, num_subcores=NS),
        scratch_types=[
            pltpu.VMEM((IK, CH), jnp.int32),
            pltpu.VMEM((CH, D), jnp.float32),
            pltpu.SemaphoreType.DMA,
        ],
    )
    return kern(tokens2d, table)


def kernel(tokens, table):
    tokens2d = tokens.reshape(NCHUNK, CH).astype(jnp.int32)
    out = _embed(tokens2d, table)
    return out.reshape(tokens.shape + (D,))


# speedup vs baseline: 10.8805x; 1.5280x over previous
import jax
import jax.numpy as jnp
from jax import lax
from jax.experimental import pallas as pl
from jax.experimental.pallas import tpu as pltpu
from jax.experimental.pallas import tpu_sc as plsc

VOCAB = 50257
D = 128
NC = 2
NS = 16
NW = NC * NS

B = 16384 * 200
CH = 128
NCHUNK = B // CH
CPW = NCHUNK // NW
IK = 80
NGROUP = CPW // IK
NBUF = 4
LA = 2


def _body(tok_hbm, tab_hbm, out_hbm,
          idx0, idx1, r0, r1, r2, r3,
          g0, g1, g2, g3, s0, s1, s2, s3, i0, i1):
    rows = (r0, r1, r2, r3)
    idxb = (idx0, idx1)
    gsem = (g0, g1, g2, g3)
    ssem = (s0, s1, s2, s3)
    isem = (i0, i1)

    wid = lax.axis_index("s") * NC + lax.axis_index("c")
    chunk0 = wid * CPW

    def gwait(b):
        pltpu.make_async_copy(tab_hbm.at[pl.ds(0, CH)], rows[b], gsem[b]).wait()

    def swait(b):
        pltpu.make_async_copy(rows[b], out_hbm.at[pl.ds(0, CH)], ssem[b]).wait()

    def gstart(b, ib, row):
        pltpu.async_copy(tab_hbm.at[ib.at[row]], rows[b], gsem[b])

    def sstart(b, gchunk):
        pltpu.async_copy(rows[b], out_hbm.at[pl.ds(gchunk * CH, CH)], ssem[b])

    pltpu.sync_copy(tok_hbm.at[pl.ds(chunk0, IK)], idxb[0])

    for G in range(NGROUP):
        ib = idxb[G % 2]
        gc0 = chunk0 + G * IK
        if G > 0:
            pltpu.make_async_copy(tok_hbm.at[pl.ds(0, IK)], ib, isem[G % 2]).wait()
        if G + 1 < NGROUP:
            pltpu.async_copy(tok_hbm.at[pl.ds(gc0 + IK, IK)],
                             idxb[(G + 1) % 2], isem[(G + 1) % 2])

        for b in range(LA):
            if G > 0:
                swait(b)
            gstart(b, ib, b)

        def step(s, carry):
            for b in range(NBUF):
                g = s * NBUF + b
                f = g + LA
                fb = (b + LA) % NBUF

                if G == 0:
                    @pl.when(jnp.logical_and(f < IK, f >= NBUF))
                    def _():
                        swait(fb)
                        gstart(fb, ib, f)

                    @pl.when(f < NBUF)
                    def _():
                        gstart(fb, ib, f)
                else:
                    @pl.when(f < IK)
                    def _():
                        swait(fb)
                        gstart(fb, ib, f)

                gwait(b)
                sstart(b, gc0 + g)
            return carry

        lax.fori_loop(0, IK // NBUF, step, 0)

    for b in range(NBUF):
        swait(b)


@jax.jit
def _embed(tokens2d, table):
    kern = pl.kernel(
        _body,
        out_type=jax.ShapeDtypeStruct((B, D), jnp.float32),
        mesh=plsc.VectorSubcoreMesh(
            core_axis_name="c", subcore_axis_name="s",
            num_cores=NC, num_subcores=NS),
        scratch_types=[
            pltpu.VMEM((IK, CH), jnp.int32),
            pltpu.VMEM((IK, CH), jnp.int32),
            pltpu.VMEM((CH, D), jnp.float32),
            pltpu.VMEM((CH, D), jnp.float32),
            pltpu.VMEM((CH, D), jnp.float32),
            pltpu.VMEM((CH, D), jnp.float32),
        ] + [pltpu.SemaphoreType.DMA] * 10,
    )
    return kern(tokens2d, table)


def kernel(tokens, table):
    tokens2d = tokens.reshape(NCHUNK, CH).astype(jnp.int32)
    out = _embed(tokens2d, table)
    return out.reshape(tokens.shape + (D,))
